# Initial kernel scaffold; baseline (speedup 1.0000x reference)
import jax
import jax.numpy as jnp
from jax.experimental import pallas as pl


def kernel(all_embed, latent_emb, weight, disen_weight_att, mean_vals, heads_tensor, tails_tensor, mean_rows, mean_cols, path_nodes_3, path_nodes_5, batch_users, batch_item, neg):
    raise NotImplementedError("write your pallas kernel here")


# speedup vs baseline: 1.1445x; 1.1445x over previous
import functools

import jax
import jax.numpy as jnp
from jax import lax
from jax.experimental import pallas as pl
from jax.experimental.pallas import tpu as pltpu

N_USERS = 10000
N_ITEMS = 40000
N_NODES = 50000
EMB = 100
EMBP = 128
N_FACTORS = 4
NFP = 8
META_LEN = 8
N_EDGES = 800000
NNZ = 500000
NP3 = 100000
NP5 = 50000
NPATH = NP3 + NP5
BATCH = 4096
DECAY = 1e-05
TEMPERATURE = 0.005

_NEG = -1e30


def _path_dense_body(rows_ref, scale_ref, lat_ref, att_ref, w_ref, dw_ref, acc):
    i = pl.program_id(0)
    n = pl.num_programs(0)

    @pl.when(i == 0)
    def _init():
        acc[...] = jnp.zeros_like(acc)

    rows = rows_ref[...] * scale_ref[...]
    score = jax.lax.dot_general(rows, lat_ref[...], (((1,), (1,)), ((), ())),
                                preferred_element_type=jnp.float32)
    score = score * (1.0 / TEMPERATURE)
    mask = jnp.where(lax.broadcasted_iota(jnp.int32, score.shape, 1) < N_FACTORS,
                     0.0, _NEG)
    score = score + mask
    score = score - jnp.max(score, axis=1, keepdims=True)
    e = jnp.exp(score)
    sm = e / jnp.sum(e, axis=1, keepdims=True)
    acc[...] += jax.lax.dot_general(sm, rows, (((0,), (0,)), ((), ())),
                                    preferred_element_type=jnp.float32)

    @pl.when(i == n - 1)
    def _fin():
        att = att_ref[...]
        att = att - jnp.max(att, axis=1, keepdims=True)
        ea = jnp.exp(att)
        sma = ea / jnp.sum(ea, axis=1, keepdims=True)
        disen = jax.lax.dot_general(sma, w_ref[...], (((1,), (0,)), ((), ())),
                                    preferred_element_type=jnp.float32)
        dw = 0.4 * disen + 0.6 * acc[...]
        nrm = jnp.sqrt(jnp.sum(dw * dw, axis=1, keepdims=True))
        dw_ref[...] = dw / jnp.maximum(nrm, 1e-12)


def _path_dense(rows, scale, lat8, att8, w8):
    npad = rows.shape[0]
    blk = 2048
    grid = npad // blk
    return pl.pallas_call(
        _path_dense_body,
        grid=(grid,),
        in_specs=[
            pl.BlockSpec((blk, EMBP), lambda i: (i, 0)),
            pl.BlockSpec((blk, 1), lambda i: (i, 0)),
            pl.BlockSpec((NFP, EMBP), lambda i: (0, 0)),
            pl.BlockSpec((NFP, META_LEN), lambda i: (0, 0)),
            pl.BlockSpec((META_LEN, EMBP), lambda i: (0, 0)),
        ],
        out_specs=pl.BlockSpec((NFP, EMBP), lambda i: (0, 0)),
        out_shape=jax.ShapeDtypeStruct((NFP, EMBP), jnp.float32),
        scratch_shapes=[pltpu.VMEM((NFP, EMBP), jnp.float32)],
    )(rows, scale, lat8, att8, w8)


def _item_fin_body(sums_ref, cnt_ref, item0_ref, new_ref, res_ref):
    s = sums_ref[...]
    c = jnp.maximum(cnt_ref[...], 1.0)
    agg = s / c
    nrm = jnp.sqrt(jnp.sum(agg * agg, axis=1, keepdims=True))
    new = agg / jnp.maximum(nrm, 1e-12)
    new_ref[...] = new
    res_ref[...] = item0_ref[...] + 2.0 * new


def _item_finalize(sums, cnts, item0):
    npad = sums.shape[0]
    blk = 2000
    grid = npad // blk
    return pl.pallas_call(
        _item_fin_body,
        grid=(grid,),
        in_specs=[
            pl.BlockSpec((blk, EMBP), lambda i: (i, 0)),
            pl.BlockSpec((blk, 1), lambda i: (i, 0)),
            pl.BlockSpec((blk, EMBP), lambda i: (i, 0)),
        ],
        out_specs=[
            pl.BlockSpec((blk, EMBP), lambda i: (i, 0)),
            pl.BlockSpec((blk, EMBP), lambda i: (i, 0)),
        ],
        out_shape=[
            jax.ShapeDtypeStruct((npad, EMBP), jnp.float32),
            jax.ShapeDtypeStruct((npad, EMBP), jnp.float32),
        ],
    )(sums, cnts, item0)


def _user_fin_body(u0_ref, a1_ref, a2_ref, lat_ref, dw_ref, res_ref):
    lat = lat_ref[...]
    dw = dw_ref[...]
    colmask = jnp.where(
        lax.broadcasted_iota(jnp.int32, (u0_ref.shape[0], NFP), 1) < N_FACTORS,
        0.0, _NEG)

    def mod(u):
        score = jax.lax.dot_general(u, lat, (((1,), (1,)), ((), ())),
                                    preferred_element_type=jnp.float32)
        score = score + colmask
        score = score - jnp.max(score, axis=1, keepdims=True)
        e = jnp.exp(score)
        sm = e / jnp.sum(e, axis=1, keepdims=True)
        return 1.0 + jax.lax.dot_general(sm, dw, (((1,), (0,)), ((), ())),
                                         preferred_element_type=jnp.float32)

    def norm(x):
        nrm = jnp.sqrt(jnp.sum(x * x, axis=1, keepdims=True))
        return x / jnp.maximum(nrm, 1e-12)

    u0 = u0_ref[...]
    u1 = norm(a1_ref[...] * mod(u0))
    u2 = norm(a2_ref[...] * mod(u1))
    res_ref[...] = u0 + u1 + u2


def _user_finalize(u0, agg1, agg2, lat8, dw8):
    npad = u0.shape[0]
    blk = 2048
    grid = npad // blk
    return pl.pallas_call(
        _user_fin_body,
        grid=(grid,),
        in_specs=[
            pl.BlockSpec((blk, EMBP), lambda i: (i, 0)),
            pl.BlockSpec((blk, EMBP), lambda i: (i, 0)),
            pl.BlockSpec((blk, EMBP), lambda i: (i, 0)),
            pl.BlockSpec((NFP, EMBP), lambda i: (0, 0)),
            pl.BlockSpec((NFP, EMBP), lambda i: (0, 0)),
        ],
        out_specs=pl.BlockSpec((blk, EMBP), lambda i: (i, 0)),
        out_shape=jax.ShapeDtypeStruct((npad, EMBP), jnp.float32),
    )(u0, agg1, agg2, lat8, dw8)


def _loss_body(u_ref, p_ref, n_ref, out_ref, acc):
    i = pl.program_id(0)
    ng = pl.num_programs(0)

    @pl.when(i == 0)
    def _init():
        acc[0] = 0.0
        acc[1] = 0.0

    u = u_ref[...]
    p = p_ref[...]
    nn = n_ref[...]
    pos = jnp.sum(u * p, axis=1)
    neg = jnp.sum(u * nn, axis=1)
    x = neg - pos
    sp = jnp.maximum(x, 0.0) + jnp.log(1.0 + jnp.exp(-jnp.abs(x)))
    reg = jnp.sum(u * u) + jnp.sum(p * p) + jnp.sum(nn * nn)
    acc[0] += jnp.sum(sp)
    acc[1] += reg

    @pl.when(i == ng - 1)
    def _fin():
        val = acc[0] / BATCH + DECAY * acc[1] / 2.0 / BATCH
        out_ref[...] = jnp.full(out_ref.shape, val, jnp.float32)


def _loss(u_e, pos_e, neg_e):
    blk = 1024
    grid = BATCH // blk
    return pl.pallas_call(
        _loss_body,
        grid=(grid,),
        in_specs=[
            pl.BlockSpec((blk, EMBP), lambda i: (i, 0)),
            pl.BlockSpec((blk, EMBP), lambda i: (i, 0)),
            pl.BlockSpec((blk, EMBP), lambda i: (i, 0)),
        ],
        out_specs=pl.BlockSpec((1, 128), lambda i: (0, 0)),
        out_shape=jax.ShapeDtypeStruct((1, 128), jnp.float32),
        scratch_shapes=[pltpu.SMEM((2,), jnp.float32)],
    )(u_e, pos_e, neg_e)


def _pad_cols(x, width=EMBP):
    return jnp.pad(x, ((0, 0), (0, width - x.shape[1])))


def _pad_rows(x, rows):
    return jnp.pad(x, ((0, rows - x.shape[0]), (0, 0)))


def kernel(all_embed, latent_emb, weight, disen_weight_att, mean_vals,
           heads_tensor, tails_tensor, mean_rows, mean_cols,
           path_nodes_3, path_nodes_5, batch_users, batch_item, neg):
    f32 = jnp.float32
    lat8 = _pad_cols(jnp.pad(latent_emb, ((0, NFP - N_FACTORS), (0, 0))))
    att8 = jnp.pad(disen_weight_att, ((0, NFP - N_FACTORS), (0, 0)))
    w8 = _pad_cols(weight)

    head_idx = heads_tensor - N_USERS
    gnn = jnp.take(all_embed, tails_tensor, axis=0)
    sums = jax.ops.segment_sum(gnn, head_idx, num_segments=N_ITEMS)
    cnts = jax.ops.segment_sum(jnp.ones((N_EDGES,), f32), head_idx,
                               num_segments=N_ITEMS)
    sums_p = _pad_cols(sums)
    cnts_p = cnts[:, None]
    item0 = all_embed[N_USERS:N_USERS + N_ITEMS]
    item_new, item_res = _item_finalize(sums_p, cnts_p, _pad_cols(item0))

    p3 = jnp.take(all_embed, path_nodes_3, axis=0).reshape(NP3, 3, EMB).sum(axis=1)
    p5 = jnp.take(all_embed, path_nodes_5, axis=0).reshape(NP5, 5, EMB).sum(axis=1)
    rows = _pad_cols(jnp.concatenate([p3, p5], axis=0))
    npad = 151552
    rows = _pad_rows(rows, npad)
    scale = jnp.concatenate([
        jnp.full((NP3, 1), 1.0 / 3.0, f32),
        jnp.full((NP5, 1), 1.0 / 5.0, f32),
        jnp.zeros((npad - NPATH, 1), f32)])
    dw8 = _path_dense(rows, scale, lat8, att8, w8)

    item_tab1 = item0
    item_tab2 = item_new[:N_ITEMS, :EMB]
    w_vals = mean_vals[:, None]
    agg1 = jax.ops.segment_sum(w_vals * jnp.take(item_tab1, mean_cols, axis=0),
                               mean_rows, num_segments=N_USERS)
    agg2 = jax.ops.segment_sum(w_vals * jnp.take(item_tab2, mean_cols, axis=0),
                               mean_rows, num_segments=N_USERS)

    upad = 10240
    u0 = _pad_rows(_pad_cols(all_embed[:N_USERS]), upad)
    a1 = _pad_rows(_pad_cols(agg1), upad)
    a2 = _pad_rows(_pad_cols(agg2), upad)
    user_res = _user_finalize(u0, a1, a2, lat8, dw8)

    u_e = jnp.take(user_res, batch_users, axis=0)
    pos_e = jnp.take(item_res, batch_item - N_USERS, axis=0)
    neg_e = jnp.take(item_res, neg, axis=0)
    out = _loss(u_e, pos_e, neg_e)
    return out[0, 0]


# speedup vs baseline: 1.9075x; 1.6666x over previous
import functools

import jax
import jax.numpy as jnp
from jax import lax
from jax.experimental import pallas as pl
from jax.experimental.pallas import tpu as pltpu
from jax.experimental.pallas import tpu_sc as plsc

N_USERS = 10000
N_ITEMS = 40000
N_NODES = 50000
EMB = 100
EMBP = 128
N_FACTORS = 4
NFP = 8
META_LEN = 8
N_EDGES = 800000
NNZ = 500000
NP3 = 100000
NP5 = 50000
NPATH = NP3 + NP5
BATCH = 4096
DECAY = 1e-05
TEMPERATURE = 0.005

_NEG = -1e30


def _path_dense_body(rows_ref, scale_ref, lat_ref, att_ref, w_ref, dw_ref, acc):
    i = pl.program_id(0)
    n = pl.num_programs(0)

    @pl.when(i == 0)
    def _init():
        acc[...] = jnp.zeros_like(acc)

    rows = rows_ref[...] * scale_ref[...]
    score = jax.lax.dot_general(rows, lat_ref[...], (((1,), (1,)), ((), ())),
                                preferred_element_type=jnp.float32)
    score = score * (1.0 / TEMPERATURE)
    mask = jnp.where(lax.broadcasted_iota(jnp.int32, score.shape, 1) < N_FACTORS,
                     0.0, _NEG)
    score = score + mask
    score = score - jnp.max(score, axis=1, keepdims=True)
    e = jnp.exp(score)
    sm = e / jnp.sum(e, axis=1, keepdims=True)
    acc[...] += jax.lax.dot_general(sm, rows, (((0,), (0,)), ((), ())),
                                    preferred_element_type=jnp.float32)

    @pl.when(i == n - 1)
    def _fin():
        att = att_ref[...]
        att = att - jnp.max(att, axis=1, keepdims=True)
        ea = jnp.exp(att)
        sma = ea / jnp.sum(ea, axis=1, keepdims=True)
        disen = jax.lax.dot_general(sma, w_ref[...], (((1,), (0,)), ((), ())),
                                    preferred_element_type=jnp.float32)
        dw = 0.4 * disen + 0.6 * acc[...]
        nrm = jnp.sqrt(jnp.sum(dw * dw, axis=1, keepdims=True))
        dw_ref[...] = dw / jnp.maximum(nrm, 1e-12)


def _path_dense(rows, scale, lat8, att8, w8):
    npad = rows.shape[0]
    blk = 2048
    grid = npad // blk
    return pl.pallas_call(
        _path_dense_body,
        grid=(grid,),
        in_specs=[
            pl.BlockSpec((blk, EMBP), lambda i: (i, 0)),
            pl.BlockSpec((blk, 1), lambda i: (i, 0)),
            pl.BlockSpec((NFP, EMBP), lambda i: (0, 0)),
            pl.BlockSpec((NFP, META_LEN), lambda i: (0, 0)),
            pl.BlockSpec((META_LEN, EMBP), lambda i: (0, 0)),
        ],
        out_specs=pl.BlockSpec((NFP, EMBP), lambda i: (0, 0)),
        out_shape=jax.ShapeDtypeStruct((NFP, EMBP), jnp.float32),
        scratch_shapes=[pltpu.VMEM((NFP, EMBP), jnp.float32)],
    )(rows, scale, lat8, att8, w8)


def _item_fin_body(a0_ref, a1_ref, b0_ref, b1_ref, item0_ref, new_ref, res_ref):
    a0 = a0_ref[...]
    a1 = a1_ref[...]
    b = b0_ref[...] + b1_ref[...]
    blk = a0.shape[0]
    s = jnp.concatenate(
        [a0, a1, b[:, :20], jnp.zeros((blk, EMBP - 100), jnp.float32)],
        axis=1)
    c = jnp.maximum(b[:, 20:21], 1.0)
    agg = s / c
    nrm = jnp.sqrt(jnp.sum(agg * agg, axis=1, keepdims=True))
    new = agg / jnp.maximum(nrm, 1e-12)
    new_ref[...] = new
    res_ref[...] = item0_ref[...] + 2.0 * new


def _item_finalize(a0, a1, b0, b1, item0):
    npad = a0.shape[0]
    blk = 2000
    grid = npad // blk
    return pl.pallas_call(
        _item_fin_body,
        grid=(grid,),
        in_specs=[
            pl.BlockSpec((blk, _WA), lambda i: (i, 0)),
            pl.BlockSpec((blk, _WA), lambda i: (i, 0)),
            pl.BlockSpec((blk, _WB), lambda i: (i, 0)),
            pl.BlockSpec((blk, _WB), lambda i: (i, 0)),
            pl.BlockSpec((blk, EMBP), lambda i: (i, 0)),
        ],
        out_specs=[
            pl.BlockSpec((blk, EMBP), lambda i: (i, 0)),
            pl.BlockSpec((blk, EMBP), lambda i: (i, 0)),
        ],
        out_shape=[
            jax.ShapeDtypeStruct((npad, EMBP), jnp.float32),
            jax.ShapeDtypeStruct((npad, EMBP), jnp.float32),
        ],
    )(a0, a1, b0, b1, item0)


def _user_fin_body(u0_ref, a1_ref, a2_ref, lat_ref, dw_ref, res_ref):
    lat = lat_ref[...]
    dw = dw_ref[...]
    colmask = jnp.where(
        lax.broadcasted_iota(jnp.int32, (u0_ref.shape[0], NFP), 1) < N_FACTORS,
        0.0, _NEG)

    def mod(u):
        score = jax.lax.dot_general(u, lat, (((1,), (1,)), ((), ())),
                                    preferred_element_type=jnp.float32)
        score = score + colmask
        score = score - jnp.max(score, axis=1, keepdims=True)
        e = jnp.exp(score)
        sm = e / jnp.sum(e, axis=1, keepdims=True)
        return 1.0 + jax.lax.dot_general(sm, dw, (((1,), (0,)), ((), ())),
                                         preferred_element_type=jnp.float32)

    def norm(x):
        nrm = jnp.sqrt(jnp.sum(x * x, axis=1, keepdims=True))
        return x / jnp.maximum(nrm, 1e-12)

    u0 = u0_ref[...]
    u1 = norm(a1_ref[...] * mod(u0))
    u2 = norm(a2_ref[...] * mod(u1))
    res_ref[...] = u0 + u1 + u2


def _user_finalize(u0, agg1, agg2, lat8, dw8):
    npad = u0.shape[0]
    blk = 2048
    grid = npad // blk
    return pl.pallas_call(
        _user_fin_body,
        grid=(grid,),
        in_specs=[
            pl.BlockSpec((blk, EMBP), lambda i: (i, 0)),
            pl.BlockSpec((blk, EMBP), lambda i: (i, 0)),
            pl.BlockSpec((blk, EMBP), lambda i: (i, 0)),
            pl.BlockSpec((NFP, EMBP), lambda i: (0, 0)),
            pl.BlockSpec((NFP, EMBP), lambda i: (0, 0)),
        ],
        out_specs=pl.BlockSpec((blk, EMBP), lambda i: (i, 0)),
        out_shape=jax.ShapeDtypeStruct((npad, EMBP), jnp.float32),
    )(u0, agg1, agg2, lat8, dw8)


def _loss_body(u_ref, p_ref, n_ref, out_ref, acc):
    i = pl.program_id(0)
    ng = pl.num_programs(0)

    @pl.when(i == 0)
    def _init():
        acc[0] = 0.0
        acc[1] = 0.0

    u = u_ref[...]
    p = p_ref[...]
    nn = n_ref[...]
    pos = jnp.sum(u * p, axis=1)
    neg = jnp.sum(u * nn, axis=1)
    x = neg - pos
    sp = jnp.maximum(x, 0.0) + jnp.log(1.0 + jnp.exp(-jnp.abs(x)))
    reg = jnp.sum(u * u) + jnp.sum(p * p) + jnp.sum(nn * nn)
    acc[0] += jnp.sum(sp)
    acc[1] += reg

    @pl.when(i == ng - 1)
    def _fin():
        val = acc[0] / BATCH + DECAY * acc[1] / 2.0 / BATCH
        out_ref[...] = jnp.full(out_ref.shape, val, jnp.float32)


def _loss(u_e, pos_e, neg_e):
    blk = 1024
    grid = BATCH // blk
    return pl.pallas_call(
        _loss_body,
        grid=(grid,),
        in_specs=[
            pl.BlockSpec((blk, EMBP), lambda i: (i, 0)),
            pl.BlockSpec((blk, EMBP), lambda i: (i, 0)),
            pl.BlockSpec((blk, EMBP), lambda i: (i, 0)),
        ],
        out_specs=pl.BlockSpec((1, 128), lambda i: (0, 0)),
        out_shape=jax.ShapeDtypeStruct((1, 128), jnp.float32),
        scratch_shapes=[pltpu.SMEM((2,), jnp.float32)],
    )(u_e, pos_e, neg_e)


_E_CHK = 512
_E_PAD = 802816
_EROWS = 40192
_WA = 40
_WB = 24

_SC_MESH = dict(core_axis_name="c", subcore_axis_name="s",
                num_cores=2, num_subcores=16)


def _edge_body(tab_ref, gidx_ref, sidx_ref, zero_ref, out_ref,
               gidx_v, sidx_v, rows_v, acc, sem):
    c = lax.axis_index("c")
    s = lax.axis_index("s")
    pltpu.sync_copy(zero_ref, acc.at[pl.ds(s * 2512, 2512)])
    plsc.subcore_barrier()
    n_chunks = gidx_ref.shape[1] // 16
    chunk0 = s * n_chunks

    def body(ci, _):
        pltpu.sync_copy(gidx_ref.at[c].at[chunk0 + ci], gidx_v.at[0])
        pltpu.sync_copy(sidx_ref.at[c].at[chunk0 + ci], sidx_v.at[0])
        hs = [pltpu.async_copy(tab_ref.at[gidx_v.at[0, j]],
                               rows_v.at[pl.ds(j * 128, 128)], sem)
              for j in range(4)]
        for h in hs:
            h.wait()
        for j in range(4):
            pltpu.sync_copy(rows_v.at[pl.ds(j * 128, 128)],
                            acc.at[sidx_v.at[0, j]], add=True)
        return 0

    lax.fori_loop(0, n_chunks, body, 0)
    plsc.subcore_barrier()
    pltpu.sync_copy(acc.at[pl.ds(s * 2512, 2512)],
                    out_ref.at[c].at[pl.ds(s * 2512, 2512)])


def _edge_scatter(tab, gidx, sidx, zeros, width):
    k = functools.partial(
        pl.kernel,
        out_type=jax.ShapeDtypeStruct((2, _EROWS, width), jnp.float32),
        mesh=plsc.VectorSubcoreMesh(**_SC_MESH),
        compiler_params=pltpu.CompilerParams(use_tc_tiling_on_sc=False),
        scratch_types=[
            pltpu.VMEM((1, 4, 128), jnp.int32),
            pltpu.VMEM((1, 4, 128), jnp.int32),
            pltpu.VMEM((_E_CHK, width), jnp.float32),
            pltpu.VMEM_SHARED((_EROWS, width), jnp.float32),
            pltpu.SemaphoreType.DMA,
        ],
    )(_edge_body)
    return k(tab, gidx, sidx, zeros)


def _pad_cols(x, width=EMBP):
    return jnp.pad(x, ((0, 0), (0, width - x.shape[1])))


def _pad_rows(x, rows):
    return jnp.pad(x, ((0, rows - x.shape[0]), (0, 0)))


def kernel(all_embed, latent_emb, weight, disen_weight_att, mean_vals,
           heads_tensor, tails_tensor, mean_rows, mean_cols,
           path_nodes_3, path_nodes_5, batch_users, batch_item, neg):
    f32 = jnp.float32
    lat8 = _pad_cols(jnp.pad(latent_emb, ((0, NFP - N_FACTORS), (0, 0))))
    att8 = jnp.pad(disen_weight_att, ((0, NFP - N_FACTORS), (0, 0)))
    w8 = _pad_cols(weight)

    nck = _E_PAD // _E_CHK
    tails_p = jnp.pad(tails_tensor, (0, _E_PAD - N_EDGES))
    heads_p = jnp.pad(heads_tensor - N_USERS, (0, _E_PAD - N_EDGES),
                      constant_values=N_ITEMS)
    tabA = jnp.concatenate([all_embed[:, :_WA], all_embed[:, _WA:2 * _WA]],
                           axis=0)
    gidxA = jnp.stack([tails_p, tails_p + N_NODES]).reshape(2, nck, 4, 128)
    sidxA = jnp.stack([heads_p, heads_p]).reshape(2, nck, 4, 128)
    hA = _edge_scatter(tabA, gidxA, sidxA, jnp.zeros((2512, _WA), f32), _WA)
    ones_col = jnp.ones((N_NODES, 1), f32)
    tabB = jnp.concatenate([all_embed[:, 2 * _WA:], ones_col,
                            jnp.zeros((N_NODES, 3), f32)], axis=1)
    gidxB = tails_p.reshape(2, nck // 2, 4, 128)
    sidxB = heads_p.reshape(2, nck // 2, 4, 128)
    hB = _edge_scatter(tabB, gidxB, sidxB, jnp.zeros((2512, _WB), f32), _WB)
    item0 = all_embed[N_USERS:N_USERS + N_ITEMS]
    item_new, item_res = _item_finalize(
        hA[0, :N_ITEMS], hA[1, :N_ITEMS], hB[0, :N_ITEMS], hB[1, :N_ITEMS],
        _pad_cols(item0))

    p3 = jnp.take(all_embed, path_nodes_3, axis=0).reshape(NP3, 3, EMB).sum(axis=1)
    p5 = jnp.take(all_embed, path_nodes_5, axis=0).reshape(NP5, 5, EMB).sum(axis=1)
    rows = _pad_cols(jnp.concatenate([p3, p5], axis=0))
    npad = 151552
    rows = _pad_rows(rows, npad)
    scale = jnp.concatenate([
        jnp.full((NP3, 1), 1.0 / 3.0, f32),
        jnp.full((NP5, 1), 1.0 / 5.0, f32),
        jnp.zeros((npad - NPATH, 1), f32)])
    dw8 = _path_dense(rows, scale, lat8, att8, w8)

    item_tab1 = item0
    item_tab2 = item_new[:N_ITEMS, :EMB]
    w_vals = mean_vals[:, None]
    agg1 = jax.ops.segment_sum(w_vals * jnp.take(item_tab1, mean_cols, axis=0),
                               mean_rows, num_segments=N_USERS)
    agg2 = jax.ops.segment_sum(w_vals * jnp.take(item_tab2, mean_cols, axis=0),
                               mean_rows, num_segments=N_USERS)

    upad = 10240
    u0 = _pad_rows(_pad_cols(all_embed[:N_USERS]), upad)
    a1 = _pad_rows(_pad_cols(agg1), upad)
    a2 = _pad_rows(_pad_cols(agg2), upad)
    user_res = _user_finalize(u0, a1, a2, lat8, dw8)

    u_e = jnp.take(user_res, batch_users, axis=0)
    pos_e = jnp.take(item_res, batch_item - N_USERS, axis=0)
    neg_e = jnp.take(item_res, neg, axis=0)
    out = _loss(u_e, pos_e, neg_e)
    return out[0, 0]


# speedup vs baseline: 2.8036x; 1.4698x over previous
import functools

import jax
import jax.numpy as jnp
from jax import lax
from jax.experimental import pallas as pl
from jax.experimental.pallas import tpu as pltpu
from jax.experimental.pallas import tpu_sc as plsc

N_USERS = 10000
N_ITEMS = 40000
N_NODES = 50000
EMB = 100
EMBP = 128
N_FACTORS = 4
NFP = 8
META_LEN = 8
N_EDGES = 800000
NNZ = 500000
NP3 = 100000
NP5 = 50000
NPATH = NP3 + NP5
BATCH = 4096
DECAY = 1e-05
TEMPERATURE = 0.005

_NEG = -1e30


def _path_dense_body(rows_ref, scale_ref, lat_ref, att_ref, w_ref, dw_ref, acc):
    i = pl.program_id(0)
    n = pl.num_programs(0)

    @pl.when(i == 0)
    def _init():
        acc[...] = jnp.zeros_like(acc)

    rows = rows_ref[...] * scale_ref[...]
    score = jax.lax.dot_general(rows, lat_ref[...], (((1,), (1,)), ((), ())),
                                preferred_element_type=jnp.float32)
    score = score * (1.0 / TEMPERATURE)
    mask = jnp.where(lax.broadcasted_iota(jnp.int32, score.shape, 1) < N_FACTORS,
                     0.0, _NEG)
    score = score + mask
    score = score - jnp.max(score, axis=1, keepdims=True)
    e = jnp.exp(score)
    sm = e / jnp.sum(e, axis=1, keepdims=True)
    acc[...] += jax.lax.dot_general(sm, rows, (((0,), (0,)), ((), ())),
                                    preferred_element_type=jnp.float32)

    @pl.when(i == n - 1)
    def _fin():
        att = att_ref[...]
        att = att - jnp.max(att, axis=1, keepdims=True)
        ea = jnp.exp(att)
        sma = ea / jnp.sum(ea, axis=1, keepdims=True)
        disen = jax.lax.dot_general(sma, w_ref[...], (((1,), (0,)), ((), ())),
                                    preferred_element_type=jnp.float32)
        dw = 0.4 * disen + 0.6 * acc[...]
        nrm = jnp.sqrt(jnp.sum(dw * dw, axis=1, keepdims=True))
        dw_ref[...] = dw / jnp.maximum(nrm, 1e-12)


def _path_dense(rows, scale, lat8, att8, w8):
    npad = rows.shape[0]
    blk = 2048
    grid = npad // blk
    return pl.pallas_call(
        _path_dense_body,
        grid=(grid,),
        in_specs=[
            pl.BlockSpec((blk, EMBP), lambda i: (i, 0)),
            pl.BlockSpec((blk, 1), lambda i: (i, 0)),
            pl.BlockSpec((NFP, EMBP), lambda i: (0, 0)),
            pl.BlockSpec((NFP, META_LEN), lambda i: (0, 0)),
            pl.BlockSpec((META_LEN, EMBP), lambda i: (0, 0)),
        ],
        out_specs=pl.BlockSpec((NFP, EMBP), lambda i: (0, 0)),
        out_shape=jax.ShapeDtypeStruct((NFP, EMBP), jnp.float32),
        scratch_shapes=[pltpu.VMEM((NFP, EMBP), jnp.float32)],
    )(rows, scale, lat8, att8, w8)


def _item_fin_body(a0_ref, a1_ref, b0_ref, b1_ref, item0_ref, new_ref, res_ref):
    a0 = a0_ref[...]
    a1 = a1_ref[...]
    b = b0_ref[...] + b1_ref[...]
    blk = a0.shape[0]
    s = jnp.concatenate(
        [a0, a1, b[:, :20], jnp.zeros((blk, EMBP - 100), jnp.float32)],
        axis=1)
    c = jnp.maximum(b[:, 20:21], 1.0)
    agg = s / c
    nrm = jnp.sqrt(jnp.sum(agg * agg, axis=1, keepdims=True))
    new = agg / jnp.maximum(nrm, 1e-12)
    new_ref[...] = new
    res_ref[...] = item0_ref[...] + 2.0 * new


def _item_finalize(a0, a1, b0, b1, item0):
    npad = a0.shape[0]
    blk = 2000
    grid = npad // blk
    return pl.pallas_call(
        _item_fin_body,
        grid=(grid,),
        in_specs=[
            pl.BlockSpec((blk, _WA), lambda i: (i, 0)),
            pl.BlockSpec((blk, _WA), lambda i: (i, 0)),
            pl.BlockSpec((blk, _WB), lambda i: (i, 0)),
            pl.BlockSpec((blk, _WB), lambda i: (i, 0)),
            pl.BlockSpec((blk, EMBP), lambda i: (i, 0)),
        ],
        out_specs=[
            pl.BlockSpec((blk, EMBP), lambda i: (i, 0)),
            pl.BlockSpec((blk, EMBP), lambda i: (i, 0)),
        ],
        out_shape=[
            jax.ShapeDtypeStruct((npad, EMBP), jnp.float32),
            jax.ShapeDtypeStruct((npad, EMBP), jnp.float32),
        ],
    )(a0, a1, b0, b1, item0)


def _user_fin_body(u0_ref, a1_ref, a2_ref, lat_ref, dw_ref, res_ref):
    lat = lat_ref[...]
    dw = dw_ref[...]
    colmask = jnp.where(
        lax.broadcasted_iota(jnp.int32, (u0_ref.shape[0], NFP), 1) < N_FACTORS,
        0.0, _NEG)

    def mod(u):
        score = jax.lax.dot_general(u, lat, (((1,), (1,)), ((), ())),
                                    preferred_element_type=jnp.float32)
        score = score + colmask
        score = score - jnp.max(score, axis=1, keepdims=True)
        e = jnp.exp(score)
        sm = e / jnp.sum(e, axis=1, keepdims=True)
        return 1.0 + jax.lax.dot_general(sm, dw, (((1,), (0,)), ((), ())),
                                         preferred_element_type=jnp.float32)

    def norm(x):
        nrm = jnp.sqrt(jnp.sum(x * x, axis=1, keepdims=True))
        return x / jnp.maximum(nrm, 1e-12)

    u0 = u0_ref[...]
    u1 = norm(a1_ref[...] * mod(u0))
    u2 = norm(a2_ref[...] * mod(u1))
    res_ref[...] = u0 + u1 + u2


def _user_finalize(u0, agg1, agg2, lat8, dw8):
    npad = u0.shape[0]
    blk = 2048
    grid = npad // blk
    return pl.pallas_call(
        _user_fin_body,
        grid=(grid,),
        in_specs=[
            pl.BlockSpec((blk, EMBP), lambda i: (i, 0)),
            pl.BlockSpec((blk, EMBP), lambda i: (i, 0)),
            pl.BlockSpec((blk, EMBP), lambda i: (i, 0)),
            pl.BlockSpec((NFP, EMBP), lambda i: (0, 0)),
            pl.BlockSpec((NFP, EMBP), lambda i: (0, 0)),
        ],
        out_specs=pl.BlockSpec((blk, EMBP), lambda i: (i, 0)),
        out_shape=jax.ShapeDtypeStruct((npad, EMBP), jnp.float32),
    )(u0, agg1, agg2, lat8, dw8)


def _loss_body(u_ref, p_ref, n_ref, out_ref, acc):
    i = pl.program_id(0)
    ng = pl.num_programs(0)

    @pl.when(i == 0)
    def _init():
        acc[0] = 0.0
        acc[1] = 0.0

    u = u_ref[...]
    p = p_ref[...]
    nn = n_ref[...]
    pos = jnp.sum(u * p, axis=1)
    neg = jnp.sum(u * nn, axis=1)
    x = neg - pos
    sp = jnp.maximum(x, 0.0) + jnp.log(1.0 + jnp.exp(-jnp.abs(x)))
    reg = jnp.sum(u * u) + jnp.sum(p * p) + jnp.sum(nn * nn)
    acc[0] += jnp.sum(sp)
    acc[1] += reg

    @pl.when(i == ng - 1)
    def _fin():
        val = acc[0] / BATCH + DECAY * acc[1] / 2.0 / BATCH
        out_ref[...] = jnp.full(out_ref.shape, val, jnp.float32)


def _loss(u_e, pos_e, neg_e):
    blk = 1024
    grid = BATCH // blk
    return pl.pallas_call(
        _loss_body,
        grid=(grid,),
        in_specs=[
            pl.BlockSpec((blk, EMBP), lambda i: (i, 0)),
            pl.BlockSpec((blk, EMBP), lambda i: (i, 0)),
            pl.BlockSpec((blk, EMBP), lambda i: (i, 0)),
        ],
        out_specs=pl.BlockSpec((1, 128), lambda i: (0, 0)),
        out_shape=jax.ShapeDtypeStruct((1, 128), jnp.float32),
        scratch_shapes=[pltpu.SMEM((2,), jnp.float32)],
    )(u_e, pos_e, neg_e)


_E_CHK = 512
_E_PAD = 802816
_EROWS = 40192
_WA = 40
_WB = 24

_SC_MESH = dict(core_axis_name="c", subcore_axis_name="s",
                num_cores=2, num_subcores=16)


def _edge_body(tab_ref, gidx_ref, sidx_ref, zero_ref, out_ref,
               gidx_v, sidx_v, rows_v, acc, sem):
    c = lax.axis_index("c")
    s = lax.axis_index("s")
    pltpu.sync_copy(zero_ref, acc.at[pl.ds(s * 2512, 2512)])
    plsc.subcore_barrier()
    n_chunks = gidx_ref.shape[1] // 16
    chunk0 = s * n_chunks

    def body(ci, _):
        pltpu.sync_copy(gidx_ref.at[c].at[chunk0 + ci], gidx_v.at[0])
        pltpu.sync_copy(sidx_ref.at[c].at[chunk0 + ci], sidx_v.at[0])
        hs = [pltpu.async_copy(tab_ref.at[gidx_v.at[0, j]],
                               rows_v.at[pl.ds(j * 128, 128)], sem)
              for j in range(4)]
        for h in hs:
            h.wait()
        for j in range(4):
            pltpu.sync_copy(rows_v.at[pl.ds(j * 128, 128)],
                            acc.at[sidx_v.at[0, j]], add=True)
        return 0

    lax.fori_loop(0, n_chunks, body, 0)
    plsc.subcore_barrier()
    pltpu.sync_copy(acc.at[pl.ds(s * 2512, 2512)],
                    out_ref.at[c].at[pl.ds(s * 2512, 2512)])


def _edge_scatter(tab, gidx, sidx, zeros, width):
    k = functools.partial(
        pl.kernel,
        out_type=jax.ShapeDtypeStruct((2, _EROWS, width), jnp.float32),
        mesh=plsc.VectorSubcoreMesh(**_SC_MESH),
        compiler_params=pltpu.CompilerParams(use_tc_tiling_on_sc=False),
        scratch_types=[
            pltpu.VMEM((1, 4, 128), jnp.int32),
            pltpu.VMEM((1, 4, 128), jnp.int32),
            pltpu.VMEM((_E_CHK, width), jnp.float32),
            pltpu.VMEM_SHARED((_EROWS, width), jnp.float32),
            pltpu.SemaphoreType.DMA,
        ],
    )(_edge_body)
    return k(tab, gidx, sidx, zeros)


_U_CHK = 512
_U_PAD = 507904
_UW = 112
_UROWS = 10240


def _user_body(tab_ref, gidx_ref, sidx_ref, vals_ref, zero_ref, out_ref,
               gidx_v, sidx_v, vals_v, rows_v, acc, sem):
    c = lax.axis_index("c")
    s = lax.axis_index("s")
    pltpu.sync_copy(zero_ref, acc.at[pl.ds(s * 640, 640)])
    plsc.subcore_barrier()
    n_chunks = gidx_ref.shape[1] // 16
    chunk0 = s * n_chunks

    def body(ci, _):
        ck = chunk0 + ci
        pltpu.sync_copy(gidx_ref.at[c].at[ck], gidx_v.at[0])
        pltpu.sync_copy(sidx_ref.at[c].at[ck], sidx_v.at[0])
        pltpu.sync_copy(vals_ref.at[ck], vals_v)
        hs = [pltpu.async_copy(tab_ref.at[gidx_v.at[0, j]],
                               rows_v.at[pl.ds(j * 128, 128)], sem)
              for j in range(4)]
        for h in hs:
            h.wait()

        dn = lax.GatherDimensionNumbers(offset_dims=(),
                                        collapsed_slice_dims=(0,),
                                        start_index_map=(0,))

        def scale_group(g, _):
            vv = vals_v[pl.ds(g * 16, 16)]
            for j in range(16):
                bv = lax.gather(vv, jnp.full((16, 1), j, jnp.int32), dn, (1,),
                                mode=lax.GatherScatterMode.PROMISE_IN_BOUNDS)
                e = g * 16 + j
                for k in range(_UW // 16):
                    rows_v[e, pl.ds(k * 16, 16)] = (
                        rows_v[e, pl.ds(k * 16, 16)] * bv)
            return 0

        lax.fori_loop(0, _U_CHK // 16, scale_group, 0)
        for j in range(4):
            pltpu.sync_copy(rows_v.at[pl.ds(j * 128, 128)],
                            acc.at[sidx_v.at[0, j]], add=True)
        return 0

    lax.fori_loop(0, n_chunks, body, 0)
    plsc.subcore_barrier()
    pltpu.sync_copy(acc.at[pl.ds(s * 640, 640)],
                    out_ref.at[c].at[pl.ds(s * 640, 640)])


def _user_scatter(tab, gidx, sidx, vals, zeros):
    k = functools.partial(
        pl.kernel,
        out_type=jax.ShapeDtypeStruct((2, _UROWS, _UW), jnp.float32),
        mesh=plsc.VectorSubcoreMesh(**_SC_MESH),
        compiler_params=pltpu.CompilerParams(use_tc_tiling_on_sc=False),
        scratch_types=[
            pltpu.VMEM((1, 4, 128), jnp.int32),
            pltpu.VMEM((1, 4, 128), jnp.int32),
            pltpu.VMEM((_U_CHK,), jnp.float32),
            pltpu.VMEM((_U_CHK, _UW), jnp.float32),
            pltpu.VMEM_SHARED((_UROWS, _UW), jnp.float32),
            pltpu.SemaphoreType.DMA,
        ],
    )(_user_body)
    return k(tab, gidx, sidx, vals, zeros)


def _pad_cols(x, width=EMBP):
    return jnp.pad(x, ((0, 0), (0, width - x.shape[1])))


def _pad_rows(x, rows):
    return jnp.pad(x, ((0, rows - x.shape[0]), (0, 0)))


def kernel(all_embed, latent_emb, weight, disen_weight_att, mean_vals,
           heads_tensor, tails_tensor, mean_rows, mean_cols,
           path_nodes_3, path_nodes_5, batch_users, batch_item, neg):
    f32 = jnp.float32
    lat8 = _pad_cols(jnp.pad(latent_emb, ((0, NFP - N_FACTORS), (0, 0))))
    att8 = jnp.pad(disen_weight_att, ((0, NFP - N_FACTORS), (0, 0)))
    w8 = _pad_cols(weight)

    nck = _E_PAD // _E_CHK
    tails_p = jnp.pad(tails_tensor, (0, _E_PAD - N_EDGES))
    heads_p = jnp.pad(heads_tensor - N_USERS, (0, _E_PAD - N_EDGES),
                      constant_values=N_ITEMS)
    tabA = jnp.concatenate([all_embed[:, :_WA], all_embed[:, _WA:2 * _WA]],
                           axis=0)
    gidxA = jnp.stack([tails_p, tails_p + N_NODES]).reshape(2, nck, 4, 128)
    sidxA = jnp.stack([heads_p, heads_p]).reshape(2, nck, 4, 128)
    hA = _edge_scatter(tabA, gidxA, sidxA, jnp.zeros((2512, _WA), f32), _WA)
    ones_col = jnp.ones((N_NODES, 1), f32)
    tabB = jnp.concatenate([all_embed[:, 2 * _WA:], ones_col,
                            jnp.zeros((N_NODES, 3), f32)], axis=1)
    gidxB = tails_p.reshape(2, nck // 2, 4, 128)
    sidxB = heads_p.reshape(2, nck // 2, 4, 128)
    hB = _edge_scatter(tabB, gidxB, sidxB, jnp.zeros((2512, _WB), f32), _WB)
    item0 = all_embed[N_USERS:N_USERS + N_ITEMS]
    item_new, item_res = _item_finalize(
        hA[0, :N_ITEMS], hA[1, :N_ITEMS], hB[0, :N_ITEMS], hB[1, :N_ITEMS],
        _pad_cols(item0))

    p3 = jnp.take(all_embed, path_nodes_3, axis=0).reshape(NP3, 3, EMB).sum(axis=1)
    p5 = jnp.take(all_embed, path_nodes_5, axis=0).reshape(NP5, 5, EMB).sum(axis=1)
    rows = _pad_cols(jnp.concatenate([p3, p5], axis=0))
    npad = 151552
    rows = _pad_rows(rows, npad)
    scale = jnp.concatenate([
        jnp.full((NP3, 1), 1.0 / 3.0, f32),
        jnp.full((NP5, 1), 1.0 / 5.0, f32),
        jnp.zeros((npad - NPATH, 1), f32)])
    dw8 = _path_dense(rows, scale, lat8, att8, w8)

    tabu = jnp.concatenate([
        jnp.pad(item0, ((0, 0), (0, _UW - EMB))), item_new[:, :_UW]], axis=0)
    cols_p = jnp.pad(mean_cols, (0, _U_PAD - NNZ))
    rows_p = jnp.pad(mean_rows, (0, _U_PAD - NNZ), constant_values=N_USERS)
    vals_p = jnp.pad(mean_vals, (0, _U_PAD - NNZ))
    nuk = _U_PAD // _U_CHK
    gidxU = jnp.stack([cols_p, cols_p + N_ITEMS]).reshape(2, nuk, 4, 128)
    sidxU = jnp.stack([rows_p, rows_p]).reshape(2, nuk, 4, 128)
    valsU = vals_p.reshape(nuk, _U_CHK)
    aggs = _user_scatter(tabu, gidxU, sidxU, valsU,
                         jnp.zeros((640, _UW), f32))

    upad = 10240
    u0 = _pad_rows(_pad_cols(all_embed[:N_USERS]), upad)
    a1 = jnp.pad(aggs[0], ((0, 0), (0, EMBP - _UW)))
    a2 = jnp.pad(aggs[1], ((0, 0), (0, EMBP - _UW)))
    user_res = _user_finalize(u0, a1, a2, lat8, dw8)

    u_e = jnp.take(user_res, batch_users, axis=0)
    pos_e = jnp.take(item_res, batch_item - N_USERS, axis=0)
    neg_e = jnp.take(item_res, neg, axis=0)
    out = _loss(u_e, pos_e, neg_e)
    return out[0, 0]
